# baseline (speedup 1.0000x reference)
import jax
import jax.numpy as jnp
from jax.experimental import pallas as pl
from jax.experimental.pallas import tpu as pltpu

EMBED = 512
HIDDEN = 512
BATCH = 64
SEQ = 32
G4 = 4 * HIDDEN
GC = 2 * G4
UNROLL = 16


def _dotT(a, w):
    return jax.lax.dot_general(
        a, w, (((1,), (1,)), ((), ())), preferred_element_type=jnp.float32)


def _sel(m, z, lo):
    return jnp.where(m, z[:, G4 + lo:G4 + lo + HIDDEN], z[:, lo:lo + HIDDEN])


def _fused_kernel(tok_ref, x_ref, wih0_ref, wih1_ref, whh0_ref, whh1_ref,
                  bi0_ref, bh0_ref, bi1_ref, bh1_ref,
                  out_ref, hF_ref, cF_ref,
                  wx_scr, wh_scr, bx_scr, msk_scr, xg_scr, h_scr, c_scr):
    i = pl.program_id(0)

    @pl.when(i == 0)
    def _prep():
        h_scr[...] = jnp.zeros_like(h_scr)
        c_scr[...] = jnp.zeros_like(c_scr)
        wx_scr[:G4] = wih0_ref[...].astype(jnp.bfloat16)
        wx_scr[G4:] = wih1_ref[...].astype(jnp.bfloat16)
        H = HIDDEN
        wh_scr[0 * H:1 * H] = whh0_ref[0 * H:1 * H].astype(jnp.bfloat16)
        wh_scr[1 * H:2 * H] = whh1_ref[0 * H:1 * H].astype(jnp.bfloat16)
        wh_scr[2 * H:3 * H] = whh0_ref[2 * H:3 * H].astype(jnp.bfloat16)
        wh_scr[3 * H:4 * H] = whh1_ref[2 * H:3 * H].astype(jnp.bfloat16)
        wh_scr[4 * H:5 * H] = whh0_ref[1 * H:2 * H].astype(jnp.bfloat16)
        wh_scr[5 * H:6 * H] = whh1_ref[1 * H:2 * H].astype(jnp.bfloat16)
        wh_scr[6 * H:7 * H] = whh0_ref[3 * H:4 * H].astype(jnp.bfloat16)
        wh_scr[7 * H:8 * H] = whh1_ref[3 * H:4 * H].astype(jnp.bfloat16)
        bx_scr[:, :G4] = bi0_ref[...] + bh0_ref[...]
        bx_scr[:, G4:] = bi1_ref[...] + bh1_ref[...]
        par = (tok_ref[...] % 2).astype(jnp.float32)
        msk_scr[...] = jnp.transpose(par, (1, 0)).reshape(SEQ, BATCH, 1)

    def mask(k):
        return msk_scr[i * UNROLL + k] > 0.5

    x = x_ref[...].reshape(UNROLL * BATCH, EMBED).astype(jnp.bfloat16)
    xg = (_dotT(x, wx_scr[...]) + bx_scr[...]).reshape(UNROLL, BATCH, GC)
    for k in range(UNROLL):
        mk = mask(k)
        xg_scr[k] = jnp.concatenate(
            [_sel(mk, xg[k], 0), _sel(mk, xg[k], HIDDEN),
             _sel(mk, xg[k], 2 * HIDDEN), _sel(mk, xg[k], 3 * HIDDEN)], axis=1)

    def half_sel(m, z, lo):
        return jnp.where(m, z[:, lo + HIDDEN:lo + 2 * HIDDEN],
                         z[:, lo:lo + HIDDEN])

    h = h_scr[...]
    c = c_scr[...]
    for k in range(UNROLL):
        hb = h.astype(jnp.bfloat16)
        d_ig = _dotT(hb, wh_scr[:G4])
        d_fo = _dotT(hb, wh_scr[G4:])
        m = mask(k)
        xk = xg_scr[k]
        gi = xk[:, 0 * HIDDEN:1 * HIDDEN] + half_sel(m, d_ig, 0)
        gg = xk[:, 2 * HIDDEN:3 * HIDDEN] + half_sel(m, d_ig, 2 * HIDDEN)
        pig = jax.nn.sigmoid(gi) * jnp.tanh(gg)
        gf = xk[:, 1 * HIDDEN:2 * HIDDEN] + half_sel(m, d_fo, 0)
        go = xk[:, 3 * HIDDEN:4 * HIDDEN] + half_sel(m, d_fo, 2 * HIDDEN)

        c = jax.nn.sigmoid(gf) * c + pig
        h = jax.nn.sigmoid(go) * jnp.tanh(c)
        out_ref[k] = h

    h_scr[...] = h
    c_scr[...] = c
    hF_ref[...] = h
    cF_ref[...] = c


def kernel(input, input_embed, W_ih0, W_hh0, b_ih0, b_hh0, W_ih1, W_hh1, b_ih1, b_hh1):
    resident = lambda shape: pl.BlockSpec(shape, lambda t: tuple(0 for _ in shape))

    out, hF, cF = pl.pallas_call(
        _fused_kernel,
        grid=(SEQ // UNROLL,),
        in_specs=[
            resident((BATCH, SEQ)),
            pl.BlockSpec((UNROLL, BATCH, EMBED), lambda i: (i, 0, 0)),
            resident((G4, EMBED)),
            resident((G4, EMBED)),
            resident((G4, HIDDEN)),
            resident((G4, HIDDEN)),
            resident((1, G4)),
            resident((1, G4)),
            resident((1, G4)),
            resident((1, G4)),
        ],
        out_specs=[
            pl.BlockSpec((UNROLL, BATCH, HIDDEN), lambda i: (i, 0, 0)),
            resident((BATCH, HIDDEN)),
            resident((BATCH, HIDDEN)),
        ],
        out_shape=[
            jax.ShapeDtypeStruct((SEQ, BATCH, HIDDEN), jnp.float32),
            jax.ShapeDtypeStruct((BATCH, HIDDEN), jnp.float32),
            jax.ShapeDtypeStruct((BATCH, HIDDEN), jnp.float32),
        ],
        scratch_shapes=[
            pltpu.VMEM((GC, EMBED), jnp.bfloat16),
            pltpu.VMEM((GC, HIDDEN), jnp.bfloat16),
            pltpu.VMEM((1, GC), jnp.float32),
            pltpu.VMEM((SEQ, BATCH, 1), jnp.float32),
            pltpu.VMEM((UNROLL, BATCH, G4), jnp.float32),
            pltpu.VMEM((BATCH, HIDDEN), jnp.float32),
            pltpu.VMEM((BATCH, HIDDEN), jnp.float32),
        ],
    )(input, input_embed, W_ih0, W_ih1, W_hh0, W_hh1,
      b_ih0.reshape(1, G4), b_hh0.reshape(1, G4),
      b_ih1.reshape(1, G4), b_hh1.reshape(1, G4))

    return out, (hF, cF)


# speedup vs baseline: 1.0111x; 1.0111x over previous
import jax
import jax.numpy as jnp
from jax.experimental import pallas as pl
from jax.experimental.pallas import tpu as pltpu

EMBED = 512
HIDDEN = 512
BATCH = 64
SEQ = 32
G4 = 4 * HIDDEN
GC = 2 * G4
UNROLL = 16


def _dotT(a, w):
    return jax.lax.dot_general(
        a, w, (((1,), (1,)), ((), ())), preferred_element_type=jnp.float32)


def _fused_kernel(tok_ref, x_ref, wih0_ref, wih1_ref, whh0_ref, whh1_ref,
                  bi0_ref, bh0_ref, bi1_ref, bh1_ref,
                  out_ref, hF_ref, cF_ref,
                  wx_scr, wh_scr, bx_scr, par_scr, xg_scr, h_scr, c_scr):
    i = pl.program_id(0)

    @pl.when(i == 0)
    def _prep():
        h_scr[...] = jnp.zeros_like(h_scr)
        c_scr[...] = jnp.zeros_like(c_scr)
        wx_scr[:G4] = wih0_ref[...].astype(jnp.bfloat16)
        wx_scr[G4:] = wih1_ref[...].astype(jnp.bfloat16)
        wh_scr[:G4] = whh0_ref[...].astype(jnp.bfloat16)
        wh_scr[G4:] = whh1_ref[...].astype(jnp.bfloat16)
        bx_scr[:, :G4] = bi0_ref[...] + bh0_ref[...]
        bx_scr[:, G4:] = bi1_ref[...] + bh1_ref[...]
        par_scr[...] = (tok_ref[...] % 2).astype(jnp.float32)

    x = x_ref[...].reshape(UNROLL * BATCH, EMBED).astype(jnp.bfloat16)
    xg_scr[...] = (_dotT(x, wx_scr[...]) + bx_scr[...]).reshape(UNROLL, BATCH, GC)

    lane = jax.lax.broadcasted_iota(jnp.int32, (BATCH, SEQ), 1)
    h = h_scr[...]
    c = c_scr[...]
    for k in range(UNROLL):
        g = xg_scr[k] + _dotT(h.astype(jnp.bfloat16), wh_scr[...])

        t = i * UNROLL + k
        mcol = jnp.sum(jnp.where(lane == t, par_scr[...], 0.0),
                       axis=1, keepdims=True)
        m = mcol > 0.5
        gi = jnp.where(m, g[:, 4 * HIDDEN:5 * HIDDEN], g[:, 0 * HIDDEN:1 * HIDDEN])
        gf = jnp.where(m, g[:, 5 * HIDDEN:6 * HIDDEN], g[:, 1 * HIDDEN:2 * HIDDEN])
        gg = jnp.where(m, g[:, 6 * HIDDEN:7 * HIDDEN], g[:, 2 * HIDDEN:3 * HIDDEN])
        go = jnp.where(m, g[:, 7 * HIDDEN:8 * HIDDEN], g[:, 3 * HIDDEN:4 * HIDDEN])

        c = jax.nn.sigmoid(gf) * c + jax.nn.sigmoid(gi) * jnp.tanh(gg)
        h = jax.nn.sigmoid(go) * jnp.tanh(c)
        out_ref[k] = h

    h_scr[...] = h
    c_scr[...] = c
    hF_ref[...] = h
    cF_ref[...] = c


def kernel(input, input_embed, W_ih0, W_hh0, b_ih0, b_hh0, W_ih1, W_hh1, b_ih1, b_hh1):
    resident = lambda shape: pl.BlockSpec(shape, lambda t: tuple(0 for _ in shape))

    out, hF, cF = pl.pallas_call(
        _fused_kernel,
        grid=(SEQ // UNROLL,),
        in_specs=[
            resident((BATCH, SEQ)),
            pl.BlockSpec((UNROLL, BATCH, EMBED), lambda i: (i, 0, 0)),
            resident((G4, EMBED)),
            resident((G4, EMBED)),
            resident((G4, HIDDEN)),
            resident((G4, HIDDEN)),
            resident((1, G4)),
            resident((1, G4)),
            resident((1, G4)),
            resident((1, G4)),
        ],
        out_specs=[
            pl.BlockSpec((UNROLL, BATCH, HIDDEN), lambda i: (i, 0, 0)),
            resident((BATCH, HIDDEN)),
            resident((BATCH, HIDDEN)),
        ],
        out_shape=[
            jax.ShapeDtypeStruct((SEQ, BATCH, HIDDEN), jnp.float32),
            jax.ShapeDtypeStruct((BATCH, HIDDEN), jnp.float32),
            jax.ShapeDtypeStruct((BATCH, HIDDEN), jnp.float32),
        ],
        scratch_shapes=[
            pltpu.VMEM((GC, EMBED), jnp.bfloat16),
            pltpu.VMEM((GC, HIDDEN), jnp.bfloat16),
            pltpu.VMEM((1, GC), jnp.float32),
            pltpu.VMEM((BATCH, SEQ), jnp.float32),
            pltpu.VMEM((UNROLL, BATCH, GC), jnp.float32),
            pltpu.VMEM((BATCH, HIDDEN), jnp.float32),
            pltpu.VMEM((BATCH, HIDDEN), jnp.float32),
        ],
    )(input, input_embed, W_ih0, W_ih1, W_hh0, W_hh1,
      b_ih0.reshape(1, G4), b_hh0.reshape(1, G4),
      b_ih1.reshape(1, G4), b_hh1.reshape(1, G4))

    return out, (hF, cF)
